# Initial kernel scaffold; baseline (speedup 1.0000x reference)
import jax
import jax.numpy as jnp
from jax.experimental import pallas as pl


def kernel(node_idx, edge_index, edge_type, emb, W1, Wroot1, b1, W2, Wroot2, b2):
    raise NotImplementedError("write your pallas kernel here")


# speedup vs baseline: 4.1359x; 4.1359x over previous
import functools

import jax
import jax.numpy as jnp
from jax.experimental import pallas as pl

NUM_NODES = 10000
HIDDEN = 128
NUM_REL = 8
NUM_EDGES = 320000

ROW_BLOCK = 1000


def _dense_body(x_ref, agg_ref, cnt_ref, w_ref, wroot_ref, b_ref, out_ref):
    x = x_ref[...]
    acc = jnp.dot(x, wroot_ref[...], preferred_element_type=jnp.float32,
                  precision=jax.lax.Precision.HIGHEST)
    acc = acc + b_ref[...]
    cnt = cnt_ref[...]
    recip = 1.0 / jnp.maximum(cnt, 1.0)
    for r in range(NUM_REL):
        a_r = agg_ref[:, r * HIDDEN:(r + 1) * HIDDEN] * recip[:, r:r + 1]
        acc = acc + jnp.dot(a_r, w_ref[r], preferred_element_type=jnp.float32,
                            precision=jax.lax.Precision.HIGHEST)
    out_ref[...] = jnp.maximum(acc, 0.0)


@functools.partial(jax.jit, static_argnames=())
def _dense_combine(x, agg, cnt, W, Wroot, b):
    nblk = NUM_NODES // ROW_BLOCK
    return pl.pallas_call(
        _dense_body,
        grid=(nblk,),
        in_specs=[
            pl.BlockSpec((ROW_BLOCK, HIDDEN), lambda i: (i, 0)),
            pl.BlockSpec((ROW_BLOCK, NUM_REL * HIDDEN), lambda i: (i, 0)),
            pl.BlockSpec((ROW_BLOCK, NUM_REL), lambda i: (i, 0)),
            pl.BlockSpec((NUM_REL, HIDDEN, HIDDEN), lambda i: (0, 0, 0)),
            pl.BlockSpec((HIDDEN, HIDDEN), lambda i: (0, 0)),
            pl.BlockSpec((1, HIDDEN), lambda i: (0, 0)),
        ],
        out_specs=pl.BlockSpec((ROW_BLOCK, HIDDEN), lambda i: (i, 0)),
        out_shape=jax.ShapeDtypeStruct((NUM_NODES, HIDDEN), jnp.float32),
    )(x, agg, cnt, W, Wroot, b)


def _aggregate(x, src, seg):
    msgs = jnp.take(x, src, axis=0)
    agg = jnp.zeros((NUM_NODES * NUM_REL, HIDDEN), jnp.float32).at[seg].add(msgs)
    cnt = jnp.zeros((NUM_NODES * NUM_REL,), jnp.float32).at[seg].add(1.0)
    return agg.reshape(NUM_NODES, NUM_REL * HIDDEN), cnt.reshape(NUM_NODES, NUM_REL)


def _layer(x, src, seg, W, Wroot, b):
    agg, cnt = _aggregate(x, src, seg)
    return _dense_combine(x, agg, cnt, W, Wroot, b.reshape(1, HIDDEN))


def kernel(node_idx, edge_index, edge_type, emb, W1, Wroot1, b1, W2, Wroot2, b2):
    x = jnp.take(emb, node_idx, axis=0)
    src = edge_index[0]
    seg = edge_index[1] * NUM_REL + edge_type
    x = _layer(x, src, seg, W1, Wroot1, b1)
    x = _layer(x, src, seg, W2, Wroot2, b2)
    return x


# speedup vs baseline: 5.4030x; 1.3064x over previous
import functools

import jax
import jax.numpy as jnp
from jax import lax
from jax.experimental import pallas as pl
from jax.experimental.pallas import tpu as pltpu
from jax.experimental.pallas import tpu_sc as plsc

NUM_NODES = 10000
HIDDEN = 128
NUM_REL = 8
NUM_EDGES = 320000

NC = 2
NS = 16
NW = NC * NS
EW = NUM_EDGES // NW
KB = 80
NBLK = EW // KB
PAD_NODES = 10240
ROWS_PER_TILE = PAD_NODES // NS
ZROWS = 160

ROW_BLOCK = 1000
_HI = jax.lax.Precision.HIGHEST
_LANES = HIDDEN // 16


def _sc_agg_body(h_ref, seg_ref, dst_ref, scale_ref, out_ref,
                 accum, segv, dstv, scalev, rows, zbuf, sem):
    cid = lax.axis_index("c")
    sid = lax.axis_index("s")
    wid = cid * NS + sid

    zero16 = jnp.zeros((16,), jnp.float32)

    def zrow(i, carry):
        for j in range(_LANES):
            zbuf[i, 16 * j:16 * (j + 1)] = zero16
        return carry

    lax.fori_loop(0, ZROWS, zrow, 0)
    for k in range(ROWS_PER_TILE // ZROWS):
        pltpu.sync_copy(
            zbuf, accum.at[pl.ds(sid * ROWS_PER_TILE + k * ZROWS, ZROWS)])
    plsc.subcore_barrier()

    base0 = wid * EW

    def block(b, carry):
        base = base0 + b * KB
        pltpu.sync_copy(seg_ref.at[pl.ds(base, KB)], segv)
        pltpu.sync_copy(dst_ref.at[pl.ds(base, KB)], dstv)
        pltpu.sync_copy(scale_ref.at[pl.ds(base, KB)], scalev)
        pltpu.async_copy(h_ref.at[segv], rows, sem).wait()

        def edge(e, c2):
            s = scalev[e, 0:16]
            for j in range(_LANES):
                rows[e, 16 * j:16 * (j + 1)] = rows[e, 16 * j:16 * (j + 1)] * s
            return c2

        lax.fori_loop(0, KB, edge, 0)
        pltpu.async_copy(rows, accum.at[dstv], sem, add=True).wait()
        return carry

    lax.fori_loop(0, NBLK, block, 0)

    plsc.subcore_barrier()
    pltpu.sync_copy(
        accum.at[pl.ds(sid * ROWS_PER_TILE, ROWS_PER_TILE)],
        out_ref.at[cid, pl.ds(sid * ROWS_PER_TILE, ROWS_PER_TILE)])


_sc_agg = functools.partial(
    pl.kernel,
    out_type=jax.ShapeDtypeStruct((NC, PAD_NODES, HIDDEN), jnp.float32),
    mesh=plsc.VectorSubcoreMesh(core_axis_name="c", subcore_axis_name="s",
                                num_cores=NC, num_subcores=NS),
    scratch_types=[
        pltpu.VMEM_SHARED((PAD_NODES, HIDDEN), jnp.float32),
        pltpu.VMEM((KB,), jnp.int32),
        pltpu.VMEM((KB,), jnp.int32),
        pltpu.VMEM((KB, 16), jnp.float32),
        pltpu.VMEM((KB, HIDDEN), jnp.float32),
        pltpu.VMEM((ZROWS, HIDDEN), jnp.float32),
        pltpu.SemaphoreType.DMA,
    ],
)(_sc_agg_body)


def _hmat_body(x_ref, w_ref, out_ref):
    x = x_ref[...]
    for r in range(NUM_REL):
        out_ref[:, r * HIDDEN:(r + 1) * HIDDEN] = jnp.dot(
            x, w_ref[r], preferred_element_type=jnp.float32, precision=_HI)


def _hmat(x, W):
    nblk = NUM_NODES // ROW_BLOCK
    return pl.pallas_call(
        _hmat_body,
        grid=(nblk,),
        in_specs=[
            pl.BlockSpec((ROW_BLOCK, HIDDEN), lambda i: (i, 0)),
            pl.BlockSpec((NUM_REL, HIDDEN, HIDDEN), lambda i: (0, 0, 0)),
        ],
        out_specs=pl.BlockSpec((ROW_BLOCK, NUM_REL * HIDDEN), lambda i: (i, 0)),
        out_shape=jax.ShapeDtypeStruct((NUM_NODES, NUM_REL * HIDDEN),
                                       jnp.float32),
    )(x, W)


def _combine_body(x_ref, parts_ref, wroot_ref, b_ref, out_ref):
    acc = jnp.dot(x_ref[...], wroot_ref[...],
                  preferred_element_type=jnp.float32, precision=_HI)
    acc = acc + b_ref[...] + parts_ref[0] + parts_ref[1]
    out_ref[...] = jnp.maximum(acc, 0.0)


def _combine(x, parts, Wroot, b):
    nblk = NUM_NODES // ROW_BLOCK
    return pl.pallas_call(
        _combine_body,
        grid=(nblk,),
        in_specs=[
            pl.BlockSpec((ROW_BLOCK, HIDDEN), lambda i: (i, 0)),
            pl.BlockSpec((NC, ROW_BLOCK, HIDDEN), lambda i: (0, i, 0)),
            pl.BlockSpec((HIDDEN, HIDDEN), lambda i: (0, 0)),
            pl.BlockSpec((1, HIDDEN), lambda i: (0, 0)),
        ],
        out_specs=pl.BlockSpec((ROW_BLOCK, HIDDEN), lambda i: (i, 0)),
        out_shape=jax.ShapeDtypeStruct((NUM_NODES, HIDDEN), jnp.float32),
    )(x, parts, Wroot, b)


def kernel(node_idx, edge_index, edge_type, emb, W1, Wroot1, b1, W2, Wroot2, b2):
    x = jnp.take(emb, node_idx, axis=0)
    src = edge_index[0]
    dst = edge_index[1]
    seg = src * NUM_REL + edge_type
    seg2 = dst * NUM_REL + edge_type
    cnt = jnp.zeros((NUM_NODES * NUM_REL,), jnp.float32).at[seg2].add(1.0)
    scale = 1.0 / jnp.take(cnt, seg2)
    scale = jnp.broadcast_to(scale[:, None], (NUM_EDGES, 16))
    for (W, Wroot, b) in ((W1, Wroot1, b1), (W2, Wroot2, b2)):
        h = _hmat(x, W).reshape(NUM_NODES * NUM_REL, HIDDEN)
        parts = _sc_agg(h, seg, dst, scale)
        x = _combine(x, parts, Wroot, b.reshape(1, HIDDEN))
    return x


# speedup vs baseline: 5.5751x; 1.0319x over previous
import functools

import jax
import jax.numpy as jnp
from jax import lax
from jax.experimental import pallas as pl
from jax.experimental.pallas import tpu as pltpu
from jax.experimental.pallas import tpu_sc as plsc

NUM_NODES = 10000
HIDDEN = 128
NUM_REL = 8
NUM_EDGES = 320000

NC = 2
NS = 16
NW = NC * NS
KB = 128
NBLK = 80
EW = NBLK * KB
E_PAD = NW * EW
PAD_NODES = 10240
ROWS_PER_TILE = PAD_NODES // NS
SB = 16
NSB = NBLK // SB

ROW_BLOCK = 1000
_HI = jax.lax.Precision.HIGHEST
_LANES = HIDDEN // 16


def _sc_agg_body(h_ref, seg_ref, dst_ref, scale_ref, out_ref,
                 accum, segv, dstv, scalev, rows0, rows1,
                 gsem0, gsem1, ssem0, ssem1):
    cid = lax.axis_index("c")
    sid = lax.axis_index("s")
    wid = cid * NS + sid

    zero16 = jnp.zeros((16,), jnp.float32)

    def zrow(i, c):
        for j in range(_LANES):
            rows0[i, 16 * j:16 * (j + 1)] = zero16
        return c

    lax.fori_loop(0, KB, zrow, 0)
    for k in range(ROWS_PER_TILE // KB):
        pltpu.sync_copy(
            rows0, accum.at[pl.ds(sid * ROWS_PER_TILE + k * KB, KB)])
    plsc.subcore_barrier()

    rows = (rows0, rows1)
    gsems = (gsem0, gsem1)
    ssems = (ssem0, ssem1)

    def start_gather(b, buf):
        pltpu.async_copy(h_ref.at[segv.at[b]], rows[buf], gsems[buf])

    def wait_gather(b, buf):
        pltpu.make_async_copy(h_ref.at[segv.at[b]], rows[buf],
                              gsems[buf]).wait()

    def start_scatter(b, buf):
        pltpu.async_copy(rows[buf], accum.at[dstv.at[b]], ssems[buf],
                         add=True)

    def wait_scatter(b, buf):
        pltpu.make_async_copy(rows[buf], accum.at[dstv.at[b]],
                              ssems[buf]).wait()

    def scale_block(b, buf):
        rbuf = rows[buf]

        def g_body(g, c):
            sv = scalev[b, pl.ds(g * 16, 16)]
            for l in range(16):
                s = lax.broadcast(sv[l], (16,))
                e = g * 16 + l
                for j in range(_LANES):
                    rbuf[e, 16 * j:16 * (j + 1)] = (
                        rbuf[e, 16 * j:16 * (j + 1)] * s)
            return c

        lax.fori_loop(0, KB // 16, g_body, 0)

    npairs = SB // 2

    def superblock(sb, c):
        pltpu.sync_copy(seg_ref.at[wid, pl.ds(sb * SB, SB)], segv)
        pltpu.sync_copy(dst_ref.at[wid, pl.ds(sb * SB, SB)], dstv)
        pltpu.sync_copy(scale_ref.at[wid, pl.ds(sb * SB, SB)], scalev)
        start_gather(0, 0)
        start_gather(1, 1)

        def pair(k, c2):
            b0 = 2 * k
            b1 = 2 * k + 1
            wait_gather(b0, 0)
            scale_block(b0, 0)
            start_scatter(b0, 0)
            wait_gather(b1, 1)
            scale_block(b1, 1)
            start_scatter(b1, 1)

            @pl.when(k < npairs - 1)
            def _():
                wait_scatter(b0, 0)
                start_gather(b0 + 2, 0)
                wait_scatter(b1, 1)
                start_gather(b1 + 2, 1)

            @pl.when(k == npairs - 1)
            def _():
                wait_scatter(b0, 0)
                wait_scatter(b1, 1)

            return c2

        lax.fori_loop(0, npairs, pair, 0)
        return c

    lax.fori_loop(0, NSB, superblock, 0)

    plsc.subcore_barrier()
    pltpu.sync_copy(
        accum.at[pl.ds(sid * ROWS_PER_TILE, ROWS_PER_TILE)],
        out_ref.at[cid, pl.ds(sid * ROWS_PER_TILE, ROWS_PER_TILE)])


_sc_agg = functools.partial(
    pl.kernel,
    out_type=jax.ShapeDtypeStruct((NC, PAD_NODES, HIDDEN), jnp.float32),
    mesh=plsc.VectorSubcoreMesh(core_axis_name="c", subcore_axis_name="s",
                                num_cores=NC, num_subcores=NS),
    scratch_types=[
        pltpu.VMEM_SHARED((PAD_NODES, HIDDEN), jnp.float32),
        pltpu.VMEM((SB, KB), jnp.int32),
        pltpu.VMEM((SB, KB), jnp.int32),
        pltpu.VMEM((SB, KB), jnp.float32),
        pltpu.VMEM((KB, HIDDEN), jnp.float32),
        pltpu.VMEM((KB, HIDDEN), jnp.float32),
        pltpu.SemaphoreType.DMA,
        pltpu.SemaphoreType.DMA,
        pltpu.SemaphoreType.DMA,
        pltpu.SemaphoreType.DMA,
    ],
)(_sc_agg_body)


def _hmat_body(x_ref, w_ref, h_ref):
    x = x_ref[...]
    for r in range(NUM_REL):
        h_ref[:, r, :] = jnp.dot(x, w_ref[r],
                                 preferred_element_type=jnp.float32,
                                 precision=_HI)


def _hmat(x, W):
    nblk = NUM_NODES // ROW_BLOCK
    return pl.pallas_call(
        _hmat_body,
        grid=(nblk,),
        in_specs=[
            pl.BlockSpec((ROW_BLOCK, HIDDEN), lambda i: (i, 0)),
            pl.BlockSpec((NUM_REL, HIDDEN, HIDDEN), lambda i: (0, 0, 0)),
        ],
        out_specs=pl.BlockSpec((ROW_BLOCK, NUM_REL, HIDDEN),
                               lambda i: (i, 0, 0)),
        out_shape=jax.ShapeDtypeStruct((NUM_NODES, NUM_REL, HIDDEN),
                                       jnp.float32),
    )(x, W)


def _mid_body(x_ref, parts_ref, wroot_ref, b_ref, w_ref, x1_ref, h_ref):
    acc = jnp.dot(x_ref[...], wroot_ref[...],
                  preferred_element_type=jnp.float32, precision=_HI)
    acc = acc + b_ref[...] + parts_ref[0] + parts_ref[1]
    x1 = jnp.maximum(acc, 0.0)
    x1_ref[...] = x1
    for r in range(NUM_REL):
        h_ref[:, r, :] = jnp.dot(x1, w_ref[r],
                                 preferred_element_type=jnp.float32,
                                 precision=_HI)


def _mid(x, parts, Wroot, b, Wnext):
    nblk = NUM_NODES // ROW_BLOCK
    return pl.pallas_call(
        _mid_body,
        grid=(nblk,),
        in_specs=[
            pl.BlockSpec((ROW_BLOCK, HIDDEN), lambda i: (i, 0)),
            pl.BlockSpec((NC, ROW_BLOCK, HIDDEN), lambda i: (0, i, 0)),
            pl.BlockSpec((HIDDEN, HIDDEN), lambda i: (0, 0)),
            pl.BlockSpec((1, HIDDEN), lambda i: (0, 0)),
            pl.BlockSpec((NUM_REL, HIDDEN, HIDDEN), lambda i: (0, 0, 0)),
        ],
        out_specs=[
            pl.BlockSpec((ROW_BLOCK, HIDDEN), lambda i: (i, 0)),
            pl.BlockSpec((ROW_BLOCK, NUM_REL, HIDDEN), lambda i: (i, 0, 0)),
        ],
        out_shape=[
            jax.ShapeDtypeStruct((NUM_NODES, HIDDEN), jnp.float32),
            jax.ShapeDtypeStruct((NUM_NODES, NUM_REL, HIDDEN), jnp.float32),
        ],
    )(x, parts, Wroot, b, Wnext)


def _combine_body(x_ref, parts_ref, wroot_ref, b_ref, out_ref):
    acc = jnp.dot(x_ref[...], wroot_ref[...],
                  preferred_element_type=jnp.float32, precision=_HI)
    acc = acc + b_ref[...] + parts_ref[0] + parts_ref[1]
    out_ref[...] = jnp.maximum(acc, 0.0)


def _combine(x, parts, Wroot, b):
    nblk = NUM_NODES // ROW_BLOCK
    return pl.pallas_call(
        _combine_body,
        grid=(nblk,),
        in_specs=[
            pl.BlockSpec((ROW_BLOCK, HIDDEN), lambda i: (i, 0)),
            pl.BlockSpec((NC, ROW_BLOCK, HIDDEN), lambda i: (0, i, 0)),
            pl.BlockSpec((HIDDEN, HIDDEN), lambda i: (0, 0)),
            pl.BlockSpec((1, HIDDEN), lambda i: (0, 0)),
        ],
        out_specs=pl.BlockSpec((ROW_BLOCK, HIDDEN), lambda i: (i, 0)),
        out_shape=jax.ShapeDtypeStruct((NUM_NODES, HIDDEN), jnp.float32),
    )(x, parts, Wroot, b)


def kernel(node_idx, edge_index, edge_type, emb, W1, Wroot1, b1, W2, Wroot2, b2):
    x = jnp.take(emb, node_idx, axis=0)
    src = edge_index[0]
    dst = edge_index[1]
    seg = src * NUM_REL + edge_type
    seg2 = dst * NUM_REL + edge_type
    cnt = jnp.zeros((NUM_NODES * NUM_REL,), jnp.float32).at[seg2].add(1.0)
    scale = 1.0 / jnp.take(cnt, seg2)

    pad = E_PAD - NUM_EDGES
    zi = jnp.zeros((pad,), jnp.int32)
    seg3 = jnp.concatenate([seg, zi]).reshape(NW, NBLK, KB)
    dst3 = jnp.concatenate([dst, zi]).reshape(NW, NBLK, KB)
    scale3 = jnp.concatenate(
        [scale, jnp.zeros((pad,), jnp.float32)]).reshape(NW, NBLK, KB)

    h1 = _hmat(x, W1).reshape(NUM_NODES * NUM_REL, HIDDEN)
    parts1 = _sc_agg(h1, seg3, dst3, scale3)
    x1, h2 = _mid(x, parts1, Wroot1, b1.reshape(1, HIDDEN), W2)
    parts2 = _sc_agg(h2.reshape(NUM_NODES * NUM_REL, HIDDEN),
                     seg3, dst3, scale3)
    return _combine(x1, parts2, Wroot2, b2.reshape(1, HIDDEN))


# speedup vs baseline: 7.3187x; 1.3128x over previous
import functools

import jax
import jax.numpy as jnp
from jax import lax
from jax.experimental import pallas as pl
from jax.experimental.pallas import tpu as pltpu
from jax.experimental.pallas import tpu_sc as plsc

NUM_NODES = 10000
HIDDEN = 128
NUM_REL = 8
NUM_EDGES = 320000

NC = 2
NS = 16
NW = NC * NS
KB = 128
NBLK = 80
EW = NBLK * KB
E_PAD = NW * EW
PAD_NODES = 10240
ROWS_PER_TILE = PAD_NODES // NS
SB = 16
NSB = NBLK // SB

ROW_BLOCK = 1000
_HI = jax.lax.Precision.HIGHEST
_LANES = HIDDEN // 16


def _sc_agg_body(h_ref, seg_ref, dst_ref, scale_ref, out_ref,
                 accum, segv, dstv, scalev, rows0, rows1,
                 gsem0, gsem1, ssem0, ssem1):
    cid = lax.axis_index("c")
    sid = lax.axis_index("s")
    wid = cid * NS + sid

    zero16 = jnp.zeros((16,), jnp.float32)

    def zrow(i, c):
        for j in range(_LANES):
            rows0[i, 16 * j:16 * (j + 1)] = zero16
        return c

    lax.fori_loop(0, KB, zrow, 0)
    for k in range(ROWS_PER_TILE // KB):
        pltpu.sync_copy(
            rows0, accum.at[pl.ds(sid * ROWS_PER_TILE + k * KB, KB)])
    plsc.subcore_barrier()

    rows = (rows0, rows1)
    gsems = (gsem0, gsem1)
    ssems = (ssem0, ssem1)

    def start_gather(b, buf):
        pltpu.async_copy(h_ref.at[segv.at[b]], rows[buf], gsems[buf])

    def wait_gather(b, buf):
        pltpu.make_async_copy(h_ref.at[segv.at[b]], rows[buf],
                              gsems[buf]).wait()

    def start_scatter(b, buf):
        pltpu.async_copy(rows[buf], accum.at[dstv.at[b]], ssems[buf],
                         add=True)

    def wait_scatter(b, buf):
        pltpu.make_async_copy(rows[buf], accum.at[dstv.at[b]],
                              ssems[buf]).wait()

    def scale_block(b, buf):
        rbuf = rows[buf]

        def g_body(g, c):
            sv = scalev[b, pl.ds(g * 16, 16)]
            for l in range(16):
                s = lax.broadcast(sv[l], (16,))
                e = g * 16 + l
                for j in range(_LANES):
                    rbuf[e, 16 * j:16 * (j + 1)] = (
                        rbuf[e, 16 * j:16 * (j + 1)] * s)
            return c

        lax.fori_loop(0, KB // 16, g_body, 0)

    npairs = SB // 2

    def superblock(sb, c):
        pltpu.sync_copy(seg_ref.at[wid, pl.ds(sb * SB, SB)], segv)
        pltpu.sync_copy(dst_ref.at[wid, pl.ds(sb * SB, SB)], dstv)
        pltpu.sync_copy(scale_ref.at[wid, pl.ds(sb * SB, SB)], scalev)
        start_gather(0, 0)
        start_gather(1, 1)

        def pair(k, c2):
            b0 = 2 * k
            b1 = 2 * k + 1
            wait_gather(b0, 0)
            scale_block(b0, 0)
            start_scatter(b0, 0)
            wait_gather(b1, 1)
            scale_block(b1, 1)
            start_scatter(b1, 1)

            @pl.when(k < npairs - 1)
            def _():
                wait_scatter(b0, 0)
                start_gather(b0 + 2, 0)
                wait_scatter(b1, 1)
                start_gather(b1 + 2, 1)

            @pl.when(k == npairs - 1)
            def _():
                wait_scatter(b0, 0)
                wait_scatter(b1, 1)

            return c2

        lax.fori_loop(0, npairs, pair, 0)
        return c

    lax.fori_loop(0, NSB, superblock, 0)

    plsc.subcore_barrier()
    pltpu.sync_copy(
        accum.at[pl.ds(sid * ROWS_PER_TILE, ROWS_PER_TILE)],
        out_ref.at[cid, pl.ds(sid * ROWS_PER_TILE, ROWS_PER_TILE)])


_sc_agg = functools.partial(
    pl.kernel,
    out_type=jax.ShapeDtypeStruct((NC, PAD_NODES, HIDDEN), jnp.float32),
    mesh=plsc.VectorSubcoreMesh(core_axis_name="c", subcore_axis_name="s",
                                num_cores=NC, num_subcores=NS),
    scratch_types=[
        pltpu.VMEM_SHARED((PAD_NODES, HIDDEN), jnp.float32),
        pltpu.VMEM((SB, KB), jnp.int32),
        pltpu.VMEM((SB, KB), jnp.int32),
        pltpu.VMEM((SB, KB), jnp.float32),
        pltpu.VMEM((KB, HIDDEN), jnp.float32),
        pltpu.VMEM((KB, HIDDEN), jnp.float32),
        pltpu.SemaphoreType.DMA,
        pltpu.SemaphoreType.DMA,
        pltpu.SemaphoreType.DMA,
        pltpu.SemaphoreType.DMA,
    ],
)(_sc_agg_body)


def _hmat_body(x_ref, w_ref, h_ref):
    x = x_ref[...]
    for r in range(NUM_REL):
        h_ref[:, r, :] = jnp.dot(x, w_ref[r],
                                 preferred_element_type=jnp.float32,
                                 precision=_HI)


def _hmat(x, W):
    nblk = NUM_NODES // ROW_BLOCK
    return pl.pallas_call(
        _hmat_body,
        grid=(nblk,),
        in_specs=[
            pl.BlockSpec((ROW_BLOCK, HIDDEN), lambda i: (i, 0)),
            pl.BlockSpec((NUM_REL, HIDDEN, HIDDEN), lambda i: (0, 0, 0)),
        ],
        out_specs=pl.BlockSpec((ROW_BLOCK, NUM_REL, HIDDEN),
                               lambda i: (i, 0, 0)),
        out_shape=jax.ShapeDtypeStruct((NUM_NODES, NUM_REL, HIDDEN),
                                       jnp.float32),
    )(x, W)


def _mid_body(x_ref, parts_ref, wroot_ref, b_ref, w_ref, x1_ref, h_ref):
    acc = jnp.dot(x_ref[...], wroot_ref[...],
                  preferred_element_type=jnp.float32, precision=_HI)
    acc = acc + b_ref[...] + parts_ref[0] + parts_ref[1]
    x1 = jnp.maximum(acc, 0.0)
    x1_ref[...] = x1
    for r in range(NUM_REL):
        h_ref[:, r, :] = jnp.dot(x1, w_ref[r],
                                 preferred_element_type=jnp.float32,
                                 precision=_HI)


def _mid(x, parts, Wroot, b, Wnext):
    nblk = NUM_NODES // ROW_BLOCK
    return pl.pallas_call(
        _mid_body,
        grid=(nblk,),
        in_specs=[
            pl.BlockSpec((ROW_BLOCK, HIDDEN), lambda i: (i, 0)),
            pl.BlockSpec((NC, ROW_BLOCK, HIDDEN), lambda i: (0, i, 0)),
            pl.BlockSpec((HIDDEN, HIDDEN), lambda i: (0, 0)),
            pl.BlockSpec((1, HIDDEN), lambda i: (0, 0)),
            pl.BlockSpec((NUM_REL, HIDDEN, HIDDEN), lambda i: (0, 0, 0)),
        ],
        out_specs=[
            pl.BlockSpec((ROW_BLOCK, HIDDEN), lambda i: (i, 0)),
            pl.BlockSpec((ROW_BLOCK, NUM_REL, HIDDEN), lambda i: (i, 0, 0)),
        ],
        out_shape=[
            jax.ShapeDtypeStruct((NUM_NODES, HIDDEN), jnp.float32),
            jax.ShapeDtypeStruct((NUM_NODES, NUM_REL, HIDDEN), jnp.float32),
        ],
    )(x, parts, Wroot, b, Wnext)


def _combine_body(x_ref, parts_ref, wroot_ref, b_ref, out_ref):
    acc = jnp.dot(x_ref[...], wroot_ref[...],
                  preferred_element_type=jnp.float32, precision=_HI)
    acc = acc + b_ref[...] + parts_ref[0] + parts_ref[1]
    out_ref[...] = jnp.maximum(acc, 0.0)


def _combine(x, parts, Wroot, b):
    nblk = NUM_NODES // ROW_BLOCK
    return pl.pallas_call(
        _combine_body,
        grid=(nblk,),
        in_specs=[
            pl.BlockSpec((ROW_BLOCK, HIDDEN), lambda i: (i, 0)),
            pl.BlockSpec((NC, ROW_BLOCK, HIDDEN), lambda i: (0, i, 0)),
            pl.BlockSpec((HIDDEN, HIDDEN), lambda i: (0, 0)),
            pl.BlockSpec((1, HIDDEN), lambda i: (0, 0)),
        ],
        out_specs=pl.BlockSpec((ROW_BLOCK, HIDDEN), lambda i: (i, 0)),
        out_shape=jax.ShapeDtypeStruct((NUM_NODES, HIDDEN), jnp.float32),
    )(x, parts, Wroot, b)


def kernel(node_idx, edge_index, edge_type, emb, W1, Wroot1, b1, W2, Wroot2, b2):
    x = jnp.take(emb, node_idx, axis=0)
    src = edge_index[0]
    dst = edge_index[1]
    seg = src * NUM_REL + edge_type
    seg2 = dst * NUM_REL + edge_type
    cnt = jnp.zeros((NUM_NODES * NUM_REL,), jnp.float32).at[seg2].add(1.0)
    scale = 1.0 / jnp.take(cnt, seg2, mode="clip")

    pad = E_PAD - NUM_EDGES
    ar = jnp.arange(pad, dtype=jnp.int32)
    seg3 = jnp.concatenate(
        [seg, ar % (NUM_NODES * NUM_REL)]).reshape(NW, NBLK, KB)
    dst3 = jnp.concatenate(
        [dst, NUM_NODES + ar % (PAD_NODES - NUM_NODES)]).reshape(NW, NBLK, KB)
    scale3 = jnp.concatenate(
        [scale, jnp.zeros((pad,), jnp.float32)]).reshape(NW, NBLK, KB)

    h1 = _hmat(x, W1).reshape(NUM_NODES * NUM_REL, HIDDEN)
    parts1 = _sc_agg(h1, seg3, dst3, scale3)
    x1, h2 = _mid(x, parts1, Wroot1, b1.reshape(1, HIDDEN), W2)
    parts2 = _sc_agg(h2.reshape(NUM_NODES * NUM_REL, HIDDEN),
                     seg3, dst3, scale3)
    return _combine(x1, parts2, Wroot2, b2.reshape(1, HIDDEN))


# speedup vs baseline: 17.1368x; 2.3415x over previous
import functools

import jax
import jax.numpy as jnp
from jax import lax
from jax.experimental import pallas as pl
from jax.experimental.pallas import tpu as pltpu
from jax.experimental.pallas import tpu_sc as plsc

NUM_NODES = 10000
HIDDEN = 128
NUM_REL = 8
NUM_EDGES = 320000

NC = 2
NS = 16
NW = NC * NS
KB = 128
NBLK = 80
EW = NBLK * KB
E_PAD = NW * EW
PAD_NODES = 10240
ROWS_PER_TILE = PAD_NODES // NS
SB = 16
NSB = NBLK // SB

ROW_BLOCK = 1000
_HI = jax.lax.Precision.HIGHEST
_LANES = HIDDEN // 16


def _sc_agg_body(h_ref, seg_ref, dst_ref, cidx_ref, cnt_ref, out_ref,
                 accum, segv, dstv, cidxv, cntv0, cntv1, rows0, rows1,
                 gsem0, gsem1, ssem0, ssem1, csem0, csem1):
    cid = lax.axis_index("c")
    sid = lax.axis_index("s")
    wid = cid * NS + sid

    zero16 = jnp.zeros((16,), jnp.float32)

    def zrow(i, c):
        for j in range(_LANES):
            rows0[i, 16 * j:16 * (j + 1)] = zero16
        return c

    lax.fori_loop(0, KB, zrow, 0)
    for k in range(ROWS_PER_TILE // KB):
        pltpu.sync_copy(
            rows0, accum.at[pl.ds(sid * ROWS_PER_TILE + k * KB, KB)])
    plsc.subcore_barrier()

    rows = (rows0, rows1)
    cnts = (cntv0, cntv1)
    gsems = (gsem0, gsem1)
    ssems = (ssem0, ssem1)
    csems = (csem0, csem1)

    def start_gather(b, buf):
        pltpu.async_copy(h_ref.at[segv.at[b]], rows[buf], gsems[buf])
        pltpu.async_copy(cnt_ref.at[cidxv.at[b]], cnts[buf], csems[buf])

    def wait_gather(b, buf):
        pltpu.make_async_copy(h_ref.at[segv.at[b]], rows[buf],
                              gsems[buf]).wait()
        pltpu.make_async_copy(cnt_ref.at[cidxv.at[b]], cnts[buf],
                              csems[buf]).wait()

    def start_scatter(b, buf):
        pltpu.async_copy(rows[buf], accum.at[dstv.at[b]], ssems[buf],
                         add=True)

    def wait_scatter(b, buf):
        pltpu.make_async_copy(rows[buf], accum.at[dstv.at[b]],
                              ssems[buf]).wait()

    def scale_block(b, buf):
        rbuf = rows[buf]
        cbuf = cnts[buf]

        def g_body(g, c):
            sv = 1.0 / cbuf[pl.ds(g * 16, 16)]
            for l in range(16):
                s = lax.broadcast(sv[l], (16,))
                e = g * 16 + l
                for j in range(_LANES):
                    rbuf[e, 16 * j:16 * (j + 1)] = (
                        rbuf[e, 16 * j:16 * (j + 1)] * s)
            return c

        lax.fori_loop(0, KB // 16, g_body, 0)

    npairs = SB // 2

    def superblock(sb, c):
        pltpu.sync_copy(seg_ref.at[wid, pl.ds(sb * SB, SB)], segv)
        pltpu.sync_copy(dst_ref.at[wid, pl.ds(sb * SB, SB)], dstv)
        pltpu.sync_copy(cidx_ref.at[wid, pl.ds(sb * SB, SB)], cidxv)
        start_gather(0, 0)
        start_gather(1, 1)

        def pair(k, c2):
            b0 = 2 * k
            b1 = 2 * k + 1
            wait_gather(b0, 0)
            scale_block(b0, 0)
            start_scatter(b0, 0)
            wait_gather(b1, 1)
            scale_block(b1, 1)
            start_scatter(b1, 1)

            @pl.when(k < npairs - 1)
            def _():
                wait_scatter(b0, 0)
                start_gather(b0 + 2, 0)
                wait_scatter(b1, 1)
                start_gather(b1 + 2, 1)

            @pl.when(k == npairs - 1)
            def _():
                wait_scatter(b0, 0)
                wait_scatter(b1, 1)

            return c2

        lax.fori_loop(0, npairs, pair, 0)
        return c

    lax.fori_loop(0, NSB, superblock, 0)

    plsc.subcore_barrier()
    pltpu.sync_copy(
        accum.at[pl.ds(sid * ROWS_PER_TILE, ROWS_PER_TILE)],
        out_ref.at[cid, pl.ds(sid * ROWS_PER_TILE, ROWS_PER_TILE)])


_sc_agg = functools.partial(
    pl.kernel,
    out_type=jax.ShapeDtypeStruct((NC, PAD_NODES, HIDDEN), jnp.float32),
    mesh=plsc.VectorSubcoreMesh(core_axis_name="c", subcore_axis_name="s",
                                num_cores=NC, num_subcores=NS),
    scratch_types=[
        pltpu.VMEM_SHARED((PAD_NODES, HIDDEN), jnp.float32),
        pltpu.VMEM((SB, KB), jnp.int32),
        pltpu.VMEM((SB, KB), jnp.int32),
        pltpu.VMEM((SB, KB), jnp.int32),
        pltpu.VMEM((KB,), jnp.float32),
        pltpu.VMEM((KB,), jnp.float32),
        pltpu.VMEM((KB, HIDDEN), jnp.float32),
        pltpu.VMEM((KB, HIDDEN), jnp.float32),
        pltpu.SemaphoreType.DMA,
        pltpu.SemaphoreType.DMA,
        pltpu.SemaphoreType.DMA,
        pltpu.SemaphoreType.DMA,
        pltpu.SemaphoreType.DMA,
        pltpu.SemaphoreType.DMA,
    ],
)(_sc_agg_body)


def _hmat_body(x_ref, w_ref, h_ref):
    x = x_ref[...]
    for r in range(NUM_REL):
        h_ref[:, r, :] = jnp.dot(x, w_ref[r],
                                 preferred_element_type=jnp.float32,
                                 precision=_HI)


def _hmat(x, W):
    nblk = NUM_NODES // ROW_BLOCK
    return pl.pallas_call(
        _hmat_body,
        grid=(nblk,),
        in_specs=[
            pl.BlockSpec((ROW_BLOCK, HIDDEN), lambda i: (i, 0)),
            pl.BlockSpec((NUM_REL, HIDDEN, HIDDEN), lambda i: (0, 0, 0)),
        ],
        out_specs=pl.BlockSpec((ROW_BLOCK, NUM_REL, HIDDEN),
                               lambda i: (i, 0, 0)),
        out_shape=jax.ShapeDtypeStruct((NUM_NODES, NUM_REL, HIDDEN),
                                       jnp.float32),
    )(x, W)


def _mid_body(x_ref, parts_ref, wroot_ref, b_ref, w_ref, x1_ref, h_ref):
    acc = jnp.dot(x_ref[...], wroot_ref[...],
                  preferred_element_type=jnp.float32, precision=_HI)
    acc = acc + b_ref[...] + parts_ref[0] + parts_ref[1]
    x1 = jnp.maximum(acc, 0.0)
    x1_ref[...] = x1
    for r in range(NUM_REL):
        h_ref[:, r, :] = jnp.dot(x1, w_ref[r],
                                 preferred_element_type=jnp.float32,
                                 precision=_HI)


def _mid(x, parts, Wroot, b, Wnext):
    nblk = NUM_NODES // ROW_BLOCK
    return pl.pallas_call(
        _mid_body,
        grid=(nblk,),
        in_specs=[
            pl.BlockSpec((ROW_BLOCK, HIDDEN), lambda i: (i, 0)),
            pl.BlockSpec((NC, ROW_BLOCK, HIDDEN), lambda i: (0, i, 0)),
            pl.BlockSpec((HIDDEN, HIDDEN), lambda i: (0, 0)),
            pl.BlockSpec((1, HIDDEN), lambda i: (0, 0)),
            pl.BlockSpec((NUM_REL, HIDDEN, HIDDEN), lambda i: (0, 0, 0)),
        ],
        out_specs=[
            pl.BlockSpec((ROW_BLOCK, HIDDEN), lambda i: (i, 0)),
            pl.BlockSpec((ROW_BLOCK, NUM_REL, HIDDEN), lambda i: (i, 0, 0)),
        ],
        out_shape=[
            jax.ShapeDtypeStruct((NUM_NODES, HIDDEN), jnp.float32),
            jax.ShapeDtypeStruct((NUM_NODES, NUM_REL, HIDDEN), jnp.float32),
        ],
    )(x, parts, Wroot, b, Wnext)


def _combine_body(x_ref, parts_ref, wroot_ref, b_ref, out_ref):
    acc = jnp.dot(x_ref[...], wroot_ref[...],
                  preferred_element_type=jnp.float32, precision=_HI)
    acc = acc + b_ref[...] + parts_ref[0] + parts_ref[1]
    out_ref[...] = jnp.maximum(acc, 0.0)


def _combine(x, parts, Wroot, b):
    nblk = NUM_NODES // ROW_BLOCK
    return pl.pallas_call(
        _combine_body,
        grid=(nblk,),
        in_specs=[
            pl.BlockSpec((ROW_BLOCK, HIDDEN), lambda i: (i, 0)),
            pl.BlockSpec((NC, ROW_BLOCK, HIDDEN), lambda i: (0, i, 0)),
            pl.BlockSpec((HIDDEN, HIDDEN), lambda i: (0, 0)),
            pl.BlockSpec((1, HIDDEN), lambda i: (0, 0)),
        ],
        out_specs=pl.BlockSpec((ROW_BLOCK, HIDDEN), lambda i: (i, 0)),
        out_shape=jax.ShapeDtypeStruct((NUM_NODES, HIDDEN), jnp.float32),
    )(x, parts, Wroot, b)


def kernel(node_idx, edge_index, edge_type, emb, W1, Wroot1, b1, W2, Wroot2, b2):
    x = jnp.take(emb, node_idx, axis=0)
    src = edge_index[0]
    dst = edge_index[1]
    seg = src * NUM_REL + edge_type
    seg2 = dst * NUM_REL + edge_type
    cnt = jnp.zeros((NUM_NODES * NUM_REL,), jnp.float32).at[seg2].add(1.0)

    pad = E_PAD - NUM_EDGES
    ar = jnp.arange(pad, dtype=jnp.int32)
    seg3 = jnp.concatenate(
        [seg, ar % (NUM_NODES * NUM_REL)]).reshape(NW, NBLK, KB)
    dst3 = jnp.concatenate(
        [dst, NUM_NODES + ar % (PAD_NODES - NUM_NODES)]).reshape(NW, NBLK, KB)
    cidx3 = jnp.concatenate(
        [seg2, jnp.broadcast_to(seg2[:1], (pad,))]).reshape(NW, NBLK, KB)

    h1 = _hmat(x, W1).reshape(NUM_NODES * NUM_REL, HIDDEN)
    parts1 = _sc_agg(h1, seg3, dst3, cidx3, cnt)
    x1, h2 = _mid(x, parts1, Wroot1, b1.reshape(1, HIDDEN), W2)
    parts2 = _sc_agg(h2.reshape(NUM_NODES * NUM_REL, HIDDEN),
                     seg3, dst3, cidx3, cnt)
    return _combine(x1, parts2, Wroot2, b2.reshape(1, HIDDEN))


# speedup vs baseline: 31.6957x; 1.8496x over previous
import functools

import jax
import jax.numpy as jnp
from jax import lax
from jax.experimental import pallas as pl
from jax.experimental.pallas import tpu as pltpu
from jax.experimental.pallas import tpu_sc as plsc

NUM_NODES = 10000
HIDDEN = 128
NUM_REL = 8
NUM_EDGES = 320000

NC = 2
NS = 16
NW = NC * NS
KB = 128
NBLK = 80
EW = NBLK * KB
E_PAD = NW * EW
PAD_NODES = 10240
ROWS_PER_TILE = PAD_NODES // NS
SB = 16
NSB = NBLK // SB

ROW_BLOCK = 1000
_HI = jax.lax.Precision.HIGHEST
_LANES = HIDDEN // 16


def _sc_agg_body(h_ref, seg_ref, dst_ref, cidx_ref, cnt_ref, out_ref,
                 accum, segv, dstv, cidxv, cntv0, cntv1, rows0, rows1,
                 gsem0, gsem1, ssem0, ssem1, csem0, csem1):
    cid = lax.axis_index("c")
    sid = lax.axis_index("s")
    wid = cid * NS + sid

    zero16 = jnp.zeros((16,), jnp.float32)

    def zrow(i, c):
        for j in range(_LANES):
            rows0[i, 16 * j:16 * (j + 1)] = zero16
        return c

    lax.fori_loop(0, KB, zrow, 0)
    for k in range(ROWS_PER_TILE // KB):
        pltpu.sync_copy(
            rows0, accum.at[pl.ds(sid * ROWS_PER_TILE + k * KB, KB)])
    plsc.subcore_barrier()

    rows = (rows0, rows1)
    cnts = (cntv0, cntv1)
    gsems = (gsem0, gsem1)
    ssems = (ssem0, ssem1)
    csems = (csem0, csem1)

    def start_gather(b, buf):
        pltpu.async_copy(h_ref.at[segv.at[b]], rows[buf], gsems[buf])
        pltpu.async_copy(cnt_ref.at[cidxv.at[b]], cnts[buf], csems[buf])

    def wait_gather(b, buf):
        pltpu.make_async_copy(h_ref.at[segv.at[b]], rows[buf],
                              gsems[buf]).wait()
        pltpu.make_async_copy(cnt_ref.at[cidxv.at[b]], cnts[buf],
                              csems[buf]).wait()

    def start_scatter(b, buf):
        pltpu.async_copy(rows[buf], accum.at[dstv.at[b]], ssems[buf],
                         add=True)

    def wait_scatter(b, buf):
        pltpu.make_async_copy(rows[buf], accum.at[dstv.at[b]],
                              ssems[buf]).wait()

    def scale_block(b, buf):
        rbuf = rows[buf]
        cbuf = cnts[buf]

        def g_body(g, c):
            sv = 1.0 / cbuf[pl.ds(g * 16, 16)]
            for l in range(16):
                s = lax.broadcast(sv[l], (16,))
                e = g * 16 + l
                for j in range(_LANES):
                    rbuf[e, 16 * j:16 * (j + 1)] = (
                        rbuf[e, 16 * j:16 * (j + 1)] * s)
            return c

        lax.fori_loop(0, KB // 16, g_body, 0)

    npairs = SB // 2

    def superblock(sb, c):
        pltpu.sync_copy(seg_ref.at[wid, pl.ds(sb * SB, SB)], segv)
        pltpu.sync_copy(dst_ref.at[wid, pl.ds(sb * SB, SB)], dstv)
        pltpu.sync_copy(cidx_ref.at[wid, pl.ds(sb * SB, SB)], cidxv)
        start_gather(0, 0)
        start_gather(1, 1)

        def pair(k, c2):
            b0 = 2 * k
            b1 = 2 * k + 1
            wait_gather(b0, 0)
            scale_block(b0, 0)
            start_scatter(b0, 0)
            wait_gather(b1, 1)
            scale_block(b1, 1)
            start_scatter(b1, 1)

            @pl.when(k < npairs - 1)
            def _():
                wait_scatter(b0, 0)
                start_gather(b0 + 2, 0)
                wait_scatter(b1, 1)
                start_gather(b1 + 2, 1)

            @pl.when(k == npairs - 1)
            def _():
                wait_scatter(b0, 0)
                wait_scatter(b1, 1)

            return c2

        lax.fori_loop(0, npairs, pair, 0)
        return c

    lax.fori_loop(0, NSB, superblock, 0)

    plsc.subcore_barrier()
    pltpu.sync_copy(
        accum.at[pl.ds(sid * ROWS_PER_TILE, ROWS_PER_TILE)],
        out_ref.at[cid, pl.ds(sid * ROWS_PER_TILE, ROWS_PER_TILE)])


_sc_agg = functools.partial(
    pl.kernel,
    out_type=jax.ShapeDtypeStruct((NC, PAD_NODES, HIDDEN), jnp.float32),
    mesh=plsc.VectorSubcoreMesh(core_axis_name="c", subcore_axis_name="s",
                                num_cores=NC, num_subcores=NS),
    scratch_types=[
        pltpu.VMEM_SHARED((PAD_NODES, HIDDEN), jnp.float32),
        pltpu.VMEM((SB, KB), jnp.int32),
        pltpu.VMEM((SB, KB), jnp.int32),
        pltpu.VMEM((SB, KB), jnp.int32),
        pltpu.VMEM((KB,), jnp.float32),
        pltpu.VMEM((KB,), jnp.float32),
        pltpu.VMEM((KB, HIDDEN), jnp.float32),
        pltpu.VMEM((KB, HIDDEN), jnp.float32),
        pltpu.SemaphoreType.DMA,
        pltpu.SemaphoreType.DMA,
        pltpu.SemaphoreType.DMA,
        pltpu.SemaphoreType.DMA,
        pltpu.SemaphoreType.DMA,
        pltpu.SemaphoreType.DMA,
    ],
)(_sc_agg_body)


def _sc_cnt_body(dst_ref, typ_ref, out_ref,
                 cacc, dstv, typv, rbuf0, rbuf1, zbuf, ssem0, ssem1):
    cid = lax.axis_index("c")
    sid = lax.axis_index("s")
    wid = cid * NS + sid

    zero16 = jnp.zeros((16,), jnp.float32)

    def zrow(i, c):
        zbuf[i, 0:16] = zero16
        return c

    lax.fori_loop(0, KB, zrow, 0)
    for k in range(ROWS_PER_TILE // KB):
        pltpu.sync_copy(
            zbuf, cacc.at[pl.ds(sid * ROWS_PER_TILE + k * KB, KB)])
    plsc.subcore_barrier()

    rbufs = (rbuf0, rbuf1)
    ssems = (ssem0, ssem1)
    iota16 = lax.iota(jnp.int32, 16)

    def start_scatter(b, buf):
        pltpu.async_copy(rbufs[buf], cacc.at[dstv.at[b]], ssems[buf],
                         add=True)

    def wait_scatter(b, buf):
        pltpu.make_async_copy(rbufs[buf], cacc.at[dstv.at[b]],
                              ssems[buf]).wait()

    def fill_block(b, buf):
        rb = rbufs[buf]

        def g_body(g, c):
            tv = typv[b, pl.ds(g * 16, 16)]
            for l in range(16):
                t = lax.broadcast(tv[l], (16,))
                e = g * 16 + l
                rb[e, 0:16] = jnp.where(iota16 == t, 1.0, 0.0)
            return c

        lax.fori_loop(0, KB // 16, g_body, 0)

    npairs = SB // 2

    def superblock(sb, c):
        pltpu.sync_copy(dst_ref.at[wid, pl.ds(sb * SB, SB)], dstv)
        pltpu.sync_copy(typ_ref.at[wid, pl.ds(sb * SB, SB)], typv)

        def pair(k, c2):
            b0 = 2 * k
            b1 = 2 * k + 1
            fill_block(b0, 0)
            start_scatter(b0, 0)
            fill_block(b1, 1)
            start_scatter(b1, 1)
            wait_scatter(b0, 0)
            wait_scatter(b1, 1)
            return c2

        lax.fori_loop(0, npairs, pair, 0)
        return c

    lax.fori_loop(0, NSB, superblock, 0)

    plsc.subcore_barrier()
    pltpu.sync_copy(
        cacc.at[pl.ds(sid * ROWS_PER_TILE, ROWS_PER_TILE)],
        out_ref.at[cid, pl.ds(sid * ROWS_PER_TILE, ROWS_PER_TILE)])


_sc_cnt = functools.partial(
    pl.kernel,
    out_type=jax.ShapeDtypeStruct((NC, PAD_NODES, 16), jnp.float32),
    mesh=plsc.VectorSubcoreMesh(core_axis_name="c", subcore_axis_name="s",
                                num_cores=NC, num_subcores=NS),
    scratch_types=[
        pltpu.VMEM_SHARED((PAD_NODES, 16), jnp.float32),
        pltpu.VMEM((SB, KB), jnp.int32),
        pltpu.VMEM((SB, KB), jnp.int32),
        pltpu.VMEM((KB, 16), jnp.float32),
        pltpu.VMEM((KB, 16), jnp.float32),
        pltpu.VMEM((KB, 16), jnp.float32),
        pltpu.SemaphoreType.DMA,
        pltpu.SemaphoreType.DMA,
    ],
)(_sc_cnt_body)


def _hmat_body(x_ref, w_ref, h_ref):
    x = x_ref[...]
    for r in range(NUM_REL):
        h_ref[:, r, :] = jnp.dot(x, w_ref[r],
                                 preferred_element_type=jnp.float32,
                                 precision=_HI)


def _hmat(x, W):
    nblk = NUM_NODES // ROW_BLOCK
    return pl.pallas_call(
        _hmat_body,
        grid=(nblk,),
        in_specs=[
            pl.BlockSpec((ROW_BLOCK, HIDDEN), lambda i: (i, 0)),
            pl.BlockSpec((NUM_REL, HIDDEN, HIDDEN), lambda i: (0, 0, 0)),
        ],
        out_specs=pl.BlockSpec((ROW_BLOCK, NUM_REL, HIDDEN),
                               lambda i: (i, 0, 0)),
        out_shape=jax.ShapeDtypeStruct((NUM_NODES, NUM_REL, HIDDEN),
                                       jnp.float32),
    )(x, W)


def _mid_body(x_ref, parts_ref, wroot_ref, b_ref, w_ref, x1_ref, h_ref):
    acc = jnp.dot(x_ref[...], wroot_ref[...],
                  preferred_element_type=jnp.float32, precision=_HI)
    acc = acc + b_ref[...] + parts_ref[0] + parts_ref[1]
    x1 = jnp.maximum(acc, 0.0)
    x1_ref[...] = x1
    for r in range(NUM_REL):
        h_ref[:, r, :] = jnp.dot(x1, w_ref[r],
                                 preferred_element_type=jnp.float32,
                                 precision=_HI)


def _mid(x, parts, Wroot, b, Wnext):
    nblk = NUM_NODES // ROW_BLOCK
    return pl.pallas_call(
        _mid_body,
        grid=(nblk,),
        in_specs=[
            pl.BlockSpec((ROW_BLOCK, HIDDEN), lambda i: (i, 0)),
            pl.BlockSpec((NC, ROW_BLOCK, HIDDEN), lambda i: (0, i, 0)),
            pl.BlockSpec((HIDDEN, HIDDEN), lambda i: (0, 0)),
            pl.BlockSpec((1, HIDDEN), lambda i: (0, 0)),
            pl.BlockSpec((NUM_REL, HIDDEN, HIDDEN), lambda i: (0, 0, 0)),
        ],
        out_specs=[
            pl.BlockSpec((ROW_BLOCK, HIDDEN), lambda i: (i, 0)),
            pl.BlockSpec((ROW_BLOCK, NUM_REL, HIDDEN), lambda i: (i, 0, 0)),
        ],
        out_shape=[
            jax.ShapeDtypeStruct((NUM_NODES, HIDDEN), jnp.float32),
            jax.ShapeDtypeStruct((NUM_NODES, NUM_REL, HIDDEN), jnp.float32),
        ],
    )(x, parts, Wroot, b, Wnext)


def _combine_body(x_ref, parts_ref, wroot_ref, b_ref, out_ref):
    acc = jnp.dot(x_ref[...], wroot_ref[...],
                  preferred_element_type=jnp.float32, precision=_HI)
    acc = acc + b_ref[...] + parts_ref[0] + parts_ref[1]
    out_ref[...] = jnp.maximum(acc, 0.0)


def _combine(x, parts, Wroot, b):
    nblk = NUM_NODES // ROW_BLOCK
    return pl.pallas_call(
        _combine_body,
        grid=(nblk,),
        in_specs=[
            pl.BlockSpec((ROW_BLOCK, HIDDEN), lambda i: (i, 0)),
            pl.BlockSpec((NC, ROW_BLOCK, HIDDEN), lambda i: (0, i, 0)),
            pl.BlockSpec((HIDDEN, HIDDEN), lambda i: (0, 0)),
            pl.BlockSpec((1, HIDDEN), lambda i: (0, 0)),
        ],
        out_specs=pl.BlockSpec((ROW_BLOCK, HIDDEN), lambda i: (i, 0)),
        out_shape=jax.ShapeDtypeStruct((NUM_NODES, HIDDEN), jnp.float32),
    )(x, parts, Wroot, b)


def kernel(node_idx, edge_index, edge_type, emb, W1, Wroot1, b1, W2, Wroot2, b2):
    x = jnp.take(emb, node_idx, axis=0)
    src = edge_index[0]
    dst = edge_index[1]
    seg = src * NUM_REL + edge_type
    seg2 = dst * 16 + edge_type

    pad = E_PAD - NUM_EDGES
    ar = jnp.arange(pad, dtype=jnp.int32)
    seg3 = jnp.concatenate(
        [seg, ar % (NUM_NODES * NUM_REL)]).reshape(NW, NBLK, KB)
    dst3 = jnp.concatenate(
        [dst, NUM_NODES + ar % (PAD_NODES - NUM_NODES)]).reshape(NW, NBLK, KB)
    typ3 = jnp.concatenate([edge_type, ar % NUM_REL]).reshape(NW, NBLK, KB)
    cidx3 = jnp.concatenate(
        [seg2, jnp.broadcast_to(seg2[:1], (pad,))]).reshape(NW, NBLK, KB)

    cnt_parts = _sc_cnt(dst3, typ3)
    cnt = (cnt_parts[0] + cnt_parts[1]).reshape(PAD_NODES * 16)

    h1 = _hmat(x, W1).reshape(NUM_NODES * NUM_REL, HIDDEN)
    parts1 = _sc_agg(h1, seg3, dst3, cidx3, cnt)
    x1, h2 = _mid(x, parts1, Wroot1, b1.reshape(1, HIDDEN), W2)
    parts2 = _sc_agg(h2.reshape(NUM_NODES * NUM_REL, HIDDEN),
                     seg3, dst3, cidx3, cnt)
    return _combine(x1, parts2, Wroot2, b2.reshape(1, HIDDEN))
